# Initial kernel scaffold; baseline (speedup 1.0000x reference)
import jax
import jax.numpy as jnp
from jax.experimental import pallas as pl


def kernel(x, G0, G1, G2):
    raise NotImplementedError("write your pallas kernel here")


# speedup vs baseline: 11.1052x; 11.1052x over previous
import functools

import jax
import jax.numpy as jnp
from jax import lax
from jax.experimental import pallas as pl
from jax.experimental.pallas import tpu as pltpu
from jax.experimental.pallas import tpu_sc as plsc

_NI = 100
_PAIRS = _NI * _NI
_D = 128

_NC = 2
_NS = 16
_NW = _NC * _NS

_CHUNK = 128
_TB = 1024


def _h12_body(g1_ref, g2_ref, out_ref):
    out_ref[...] = jnp.dot(g1_ref[...], g2_ref[...],
                           preferred_element_type=jnp.float32)


def _make_pair_table(G1, G2):
    g1t = jnp.transpose(G1, (1, 0, 2, 3)).reshape(_NI * 16 * 4, 16)
    g2r = G2[:, :, :, 0].reshape(16, _NI * 2)
    h = pl.pallas_call(
        _h12_body,
        out_shape=jax.ShapeDtypeStruct((_NI * 16 * 4, _NI * 2), jnp.float32),
    )(g1t, g2r)
    h = h.reshape(_NI, 16, 4, _NI, 2)
    h = jnp.transpose(h, (0, 3, 1, 2, 4))
    return h.reshape(_PAIRS, _D)


def _sc_gather(xflat, table, bpw):
    B = xflat.shape[0]
    nch = bpw // _CHUNK
    mesh = plsc.VectorSubcoreMesh(core_axis_name="c", subcore_axis_name="s")

    @functools.partial(
        pl.kernel,
        mesh=mesh,
        out_type=jax.ShapeDtypeStruct((B, _D), jnp.float32),
        scratch_types=[
            pltpu.VMEM((bpw,), jnp.int32),
            pltpu.VMEM((bpw,), jnp.int32),
            pltpu.VMEM((_CHUNK, _D), jnp.float32),
            pltpu.VMEM((_CHUNK, _D), jnp.float32),
            pltpu.SemaphoreType.DMA,
            pltpu.SemaphoreType.DMA,
        ],
    )
    def gather(x_hbm, tab_hbm, out_hbm, xv, idxv, rows0, rows1, sem0, sem1):
        wid = lax.axis_index("s") * _NC + lax.axis_index("c")
        base = wid * bpw
        pltpu.sync_copy(x_hbm.at[pl.ds(base, bpw)], xv)

        def idx_body(i, carry):
            v = xv[pl.ds(i * 16, 16)]
            idxv[pl.ds(i * 16, 16)] = lax.rem(v, _PAIRS)
            return carry

        lax.fori_loop(0, bpw // 16, idx_body, 0)

        def pair_body(p, carry):
            c = p * 2
            d0 = pltpu.async_copy(
                tab_hbm.at[idxv.at[pl.ds(c * _CHUNK, _CHUNK)]], rows0, sem0)
            d1 = pltpu.async_copy(
                tab_hbm.at[idxv.at[pl.ds((c + 1) * _CHUNK, _CHUNK)]], rows1, sem1)
            d0.wait()
            pltpu.sync_copy(rows0, out_hbm.at[pl.ds(base + c * _CHUNK, _CHUNK)])
            d1.wait()
            pltpu.sync_copy(rows1, out_hbm.at[pl.ds(base + (c + 1) * _CHUNK, _CHUNK)])
            return carry

        lax.fori_loop(0, nch // 2, pair_body, 0)

    return gather(xflat, table)


def _contract_body(x_ref, hg_ref, g0_ref, out_ref):
    xr = x_ref[0]
    i0 = xr // _PAIRS
    iot = lax.broadcasted_iota(jnp.int32, (128, _TB), 0)
    oh = (iot == i0).astype(jnp.float32)
    a_t = jnp.dot(g0_ref[...], oh,
                  preferred_element_type=jnp.float32)
    hg_t = hg_ref[...].T
    outs = []
    for m0 in range(4):
        acc = a_t[m0 * 16:m0 * 16 + 1, :] * hg_t[0:8, :]
        for r1 in range(1, 16):
            acc = acc + (a_t[m0 * 16 + r1:m0 * 16 + r1 + 1, :]
                         * hg_t[r1 * 8:(r1 + 1) * 8, :])
        outs.append(acc)
    out_t = jnp.concatenate(outs, axis=0)
    out_ref[...] = out_t.T


def _contract(xflat, hg, g0m):
    B = xflat.shape[0]
    nb = B // _TB
    x3 = xflat.reshape(nb, 1, _TB)
    return pl.pallas_call(
        _contract_body,
        grid=(nb,),
        in_specs=[
            pl.BlockSpec((1, 1, _TB), lambda i: (i, 0, 0)),
            pl.BlockSpec((_TB, _D), lambda i: (i, 0)),
            pl.BlockSpec((64, 128), lambda i: (0, 0)),
        ],
        out_specs=pl.BlockSpec((_TB, 32), lambda i: (i, 0)),
        out_shape=jax.ShapeDtypeStruct((B, 32), jnp.float32),
    )(x3, hg, g0m)


def _g0_mat(G0):
    g = jnp.transpose(G0[0], (1, 2, 0)).reshape(64, _NI)
    return jnp.pad(g, ((0, 0), (0, 128 - _NI)))


def kernel(x, G0, G1, G2):
    xshape = x.shape
    xflat = x.reshape(-1)
    B = xflat.shape[0]
    bpw = B // _NW

    table = _make_pair_table(G1, G2)
    hg = _sc_gather(xflat, table, bpw)
    out = _contract(xflat, hg, _g0_mat(G0))
    return out.reshape(xshape + (32,))


# speedup vs baseline: 27.0385x; 2.4348x over previous
import functools

import jax
import jax.numpy as jnp
from jax import lax
from jax.experimental import pallas as pl
from jax.experimental.pallas import tpu as pltpu
from jax.experimental.pallas import tpu_sc as plsc

_NI = 100
_PAIRS = _NI * _NI
_D = 128

_NC = 2
_NS = 16
_NW = _NC * _NS

_CHUNK = 128
_BR = 64
_TB = _BR * 50


def _h12_body(g2m_ref, g1t_ref, out_ref):
    lhs = g2m_ref[...]
    row_m2 = lax.broadcasted_iota(jnp.int32, (32, 1), 0) % 2
    col_m2 = lax.broadcasted_iota(jnp.int32, (1, _D), 1) % 2
    for h in range(2):
        g1blk = g1t_ref[h * 64:(h + 1) * 64, :]
        gt = g1blk.T
        gr = jnp.repeat(gt, 2, axis=0)
        gc = jnp.repeat(gr, 2, axis=1)
        rhs = jnp.where(row_m2 == col_m2, gc, 0.0)
        out_ref[h * _NI:(h + 1) * _NI, :] = jnp.dot(
            lhs, rhs, preferred_element_type=jnp.float32)


def _make_pair_table(G1, G2):
    g1t = jnp.transpose(G1, (1, 0, 2, 3)).reshape(_NI * 64, 16)
    g2m = jnp.transpose(G2[:, :, :, 0], (1, 0, 2)).reshape(_NI, 32)
    return pl.pallas_call(
        _h12_body,
        grid=(_NI // 2,),
        in_specs=[
            pl.BlockSpec((_NI, 32), lambda i: (0, 0)),
            pl.BlockSpec((128, 16), lambda i: (i, 0)),
        ],
        out_specs=pl.BlockSpec((2 * _NI, _D), lambda i: (i, 0)),
        out_shape=jax.ShapeDtypeStruct((_PAIRS, _D), jnp.float32),
    )(g2m, g1t)


def _sc_gather(xflat, table, bpw):
    B = xflat.shape[0]
    nch = bpw // _CHUNK
    mesh = plsc.VectorSubcoreMesh(core_axis_name="c", subcore_axis_name="s")

    @functools.partial(
        pl.kernel,
        mesh=mesh,
        out_type=jax.ShapeDtypeStruct((B, _D), jnp.float32),
        scratch_types=[
            pltpu.VMEM((bpw,), jnp.int32),
            pltpu.VMEM((bpw,), jnp.int32),
            pltpu.VMEM((_CHUNK, _D), jnp.float32),
            pltpu.VMEM((_CHUNK, _D), jnp.float32),
            pltpu.SemaphoreType.DMA,
            pltpu.SemaphoreType.DMA,
        ],
    )
    def gather(x_hbm, tab_hbm, out_hbm, xv, idxv, rows0, rows1, sem0, sem1):
        wid = lax.axis_index("s") * _NC + lax.axis_index("c")
        base = wid * bpw
        pltpu.sync_copy(x_hbm.at[pl.ds(base, bpw)], xv)

        def idx_body(i, carry):
            v = xv[pl.ds(i * 16, 16)]
            idxv[pl.ds(i * 16, 16)] = lax.rem(v, _PAIRS)
            return carry

        lax.fori_loop(0, bpw // 16, idx_body, 0)

        def pair_body(p, carry):
            c = p * 2
            d0 = pltpu.async_copy(
                tab_hbm.at[idxv.at[pl.ds(c * _CHUNK, _CHUNK)]], rows0, sem0)
            d1 = pltpu.async_copy(
                tab_hbm.at[idxv.at[pl.ds((c + 1) * _CHUNK, _CHUNK)]], rows1, sem1)
            d0.wait()
            pltpu.sync_copy(rows0, out_hbm.at[pl.ds(base + c * _CHUNK, _CHUNK)])
            d1.wait()
            pltpu.sync_copy(rows1, out_hbm.at[pl.ds(base + (c + 1) * _CHUNK, _CHUNK)])
            return carry

        lax.fori_loop(0, nch // 2, pair_body, 0)

    return gather(xflat, table)


def _contract_body(x_ref, hg_ref, g0_ref, out_ref):
    xr = x_ref[0]
    i0 = xr // _PAIRS
    iot = lax.broadcasted_iota(jnp.int32, (128, _TB), 0)
    oh = (iot == i0).astype(jnp.float32)
    a_t = jnp.dot(g0_ref[...], oh,
                  preferred_element_type=jnp.float32)
    hg_t = hg_ref[...].T
    outs = []
    for m0 in range(4):
        acc = a_t[m0 * 16:m0 * 16 + 1, :] * hg_t[0:8, :]
        for r1 in range(1, 16):
            acc = acc + (a_t[m0 * 16 + r1:m0 * 16 + r1 + 1, :]
                         * hg_t[r1 * 8:(r1 + 1) * 8, :])
        outs.append(acc)
    out_t = jnp.concatenate(outs, axis=0)
    res = out_t.T
    for r in range(_BR):
        out_ref[r] = res[r * 50:(r + 1) * 50, :]


def _contract(xflat, hg, g0m):
    B = xflat.shape[0]
    nb = B // _TB
    x3 = xflat.reshape(nb, 1, _TB)
    return pl.pallas_call(
        _contract_body,
        grid=(nb,),
        in_specs=[
            pl.BlockSpec((1, 1, _TB), lambda i: (i, 0, 0)),
            pl.BlockSpec((_TB, _D), lambda i: (i, 0)),
            pl.BlockSpec((64, 128), lambda i: (0, 0)),
        ],
        out_specs=pl.BlockSpec((_BR, 50, 32), lambda i: (i, 0, 0)),
        out_shape=jax.ShapeDtypeStruct((B // 50, 50, 32), jnp.float32),
    )(x3, hg, g0m)


def _g0_mat(G0):
    g = jnp.transpose(G0[0], (1, 2, 0)).reshape(64, _NI)
    return jnp.pad(g, ((0, 0), (0, 128 - _NI)))


def kernel(x, G0, G1, G2):
    xshape = x.shape
    xflat = x.reshape(-1)
    B = xflat.shape[0]
    bpw = B // _NW

    table = _make_pair_table(G1, G2)
    hg = _sc_gather(xflat, table, bpw)
    out = _contract(xflat, hg, _g0_mat(G0))
    return out.reshape(xshape + (32,))


# speedup vs baseline: 28.9400x; 1.0703x over previous
import functools

import jax
import jax.numpy as jnp
from jax import lax
from jax.experimental import pallas as pl
from jax.experimental.pallas import tpu as pltpu
from jax.experimental.pallas import tpu_sc as plsc

_NI = 100
_PAIRS = _NI * _NI
_D = 128

_NC = 2
_NS = 16
_NW = _NC * _NS

_CHUNK = 128
_BR = 64
_TB = _BR * 50


_J1B = 10


def _h12_body(g2m_ref, g1t_ref, out_ref):
    lhs = g2m_ref[...]
    w = _D * _J1B
    gc = jnp.repeat(g1t_ref[...], 2, axis=0)
    row_m2 = lax.broadcasted_iota(jnp.int32, (32, 1), 0) % 2
    col_m2 = lax.broadcasted_iota(jnp.int32, (1, w), 1) % 2
    rhs = jnp.where(row_m2 == col_m2, gc, 0.0)
    res = jnp.dot(lhs, rhs, preferred_element_type=jnp.float32)
    for h in range(_J1B):
        out_ref[h * _NI:(h + 1) * _NI, :] = res[:, h * _D:(h + 1) * _D]


def _make_pair_table(G1, G2):
    g1t = jnp.repeat(
        jnp.transpose(G1, (3, 1, 0, 2)).reshape(16, _NI * 64), 2, axis=1)
    g2m = jnp.transpose(G2[:, :, :, 0], (1, 0, 2)).reshape(_NI, 32)
    return pl.pallas_call(
        _h12_body,
        grid=(_NI // _J1B,),
        in_specs=[
            pl.BlockSpec((_NI, 32), lambda i: (0, 0)),
            pl.BlockSpec((16, 128 * _J1B), lambda i: (0, i)),
        ],
        out_specs=pl.BlockSpec((_J1B * _NI, _D), lambda i: (i, 0)),
        out_shape=jax.ShapeDtypeStruct((_PAIRS, _D), jnp.float32),
    )(g2m, g1t)


def _sc_gather(xflat, table, bpw):
    B = xflat.shape[0]
    nch = bpw // _CHUNK
    mesh = plsc.VectorSubcoreMesh(core_axis_name="c", subcore_axis_name="s")

    @functools.partial(
        pl.kernel,
        mesh=mesh,
        out_type=jax.ShapeDtypeStruct((B, _D), jnp.float32),
        scratch_types=[
            pltpu.VMEM((bpw,), jnp.int32),
            pltpu.VMEM((bpw,), jnp.int32),
            pltpu.VMEM((_CHUNK, _D), jnp.float32),
            pltpu.VMEM((_CHUNK, _D), jnp.float32),
            pltpu.SemaphoreType.DMA,
            pltpu.SemaphoreType.DMA,
        ],
    )
    def gather(x_hbm, tab_hbm, out_hbm, xv, idxv, rows0, rows1, sem0, sem1):
        wid = lax.axis_index("s") * _NC + lax.axis_index("c")
        base = wid * bpw
        pltpu.sync_copy(x_hbm.at[pl.ds(base, bpw)], xv)

        def idx_body(i, carry):
            v = xv[pl.ds(i * 16, 16)]
            idxv[pl.ds(i * 16, 16)] = lax.rem(v, _PAIRS)
            return carry

        lax.fori_loop(0, bpw // 16, idx_body, 0)

        def pair_body(p, carry):
            c = p * 2
            d0 = pltpu.async_copy(
                tab_hbm.at[idxv.at[pl.ds(c * _CHUNK, _CHUNK)]], rows0, sem0)
            d1 = pltpu.async_copy(
                tab_hbm.at[idxv.at[pl.ds((c + 1) * _CHUNK, _CHUNK)]], rows1, sem1)
            d0.wait()
            pltpu.sync_copy(rows0, out_hbm.at[pl.ds(base + c * _CHUNK, _CHUNK)])
            d1.wait()
            pltpu.sync_copy(rows1, out_hbm.at[pl.ds(base + (c + 1) * _CHUNK, _CHUNK)])
            return carry

        lax.fori_loop(0, nch // 2, pair_body, 0)

    return gather(xflat, table)


def _contract_body(x_ref, hg_ref, g0_ref, out_ref):
    xr = x_ref[0]
    i0 = xr // _PAIRS
    iot = lax.broadcasted_iota(jnp.int32, (128, _TB), 0)
    oh = (iot == i0).astype(jnp.float32)
    a_t = jnp.dot(g0_ref[...], oh,
                  preferred_element_type=jnp.float32)
    hg_t = hg_ref[...].T
    outs = []
    for m0 in range(4):
        acc = a_t[m0 * 16:m0 * 16 + 1, :] * hg_t[0:8, :]
        for r1 in range(1, 16):
            acc = acc + (a_t[m0 * 16 + r1:m0 * 16 + r1 + 1, :]
                         * hg_t[r1 * 8:(r1 + 1) * 8, :])
        outs.append(acc)
    out_t = jnp.concatenate(outs, axis=0)
    res = out_t.T
    for r in range(_BR):
        out_ref[r] = res[r * 50:(r + 1) * 50, :]


def _contract(xflat, hg, g0m):
    B = xflat.shape[0]
    nb = B // _TB
    x3 = xflat.reshape(nb, 1, _TB)
    return pl.pallas_call(
        _contract_body,
        grid=(nb,),
        in_specs=[
            pl.BlockSpec((1, 1, _TB), lambda i: (i, 0, 0)),
            pl.BlockSpec((_TB, _D), lambda i: (i, 0)),
            pl.BlockSpec((64, 128), lambda i: (0, 0)),
        ],
        out_specs=pl.BlockSpec((_BR, 50, 32), lambda i: (i, 0, 0)),
        out_shape=jax.ShapeDtypeStruct((B // 50, 50, 32), jnp.float32),
    )(x3, hg, g0m)


def _g0_mat(G0):
    g = jnp.transpose(G0[0], (1, 2, 0)).reshape(64, _NI)
    return jnp.pad(g, ((0, 0), (0, 128 - _NI)))


def kernel(x, G0, G1, G2):
    xshape = x.shape
    xflat = x.reshape(-1)
    B = xflat.shape[0]
    bpw = B // _NW

    table = _make_pair_table(G1, G2)
    hg = _sc_gather(xflat, table, bpw)
    out = _contract(xflat, hg, _g0_mat(G0))
    if out.shape != xshape + (32,):
        out = out.reshape(xshape + (32,))
    return out
